# Initial kernel scaffold; baseline (speedup 1.0000x reference)
import jax
import jax.numpy as jnp
from jax.experimental import pallas as pl


def kernel(x, edge_index, W1, b1, W2, b2):
    raise NotImplementedError("write your pallas kernel here")


# speedup vs baseline: 1.7377x; 1.7377x over previous
import functools

import jax
import jax.numpy as jnp
from jax import lax
from jax.experimental import pallas as pl
from jax.experimental.pallas import tpu as pltpu
from jax.experimental.pallas import tpu_sc as plsc

N = 10000
E = 320000
NTILE = 16
NCORE = 2
EROWS = 160
EPAD = NTILE * EROWS * 128
ACC_ROWS = 10240
ZROWS = ACC_ROWS // NTILE
DEG_ROWS_PER_TILE = EPAD // (NCORE * NTILE * 128)
CH = 4

_mesh2 = plsc.VectorSubcoreMesh(core_axis_name="c", subcore_axis_name="s")
_mesh1 = plsc.VectorSubcoreMesh(core_axis_name="c", subcore_axis_name="s",
                                num_cores=1)


@functools.partial(
    pl.kernel,
    out_type=(
        jax.ShapeDtypeStruct((NCORE, ACC_ROWS, 128), jnp.float32),
        jax.ShapeDtypeStruct((NCORE, ACC_ROWS, 128), jnp.float32),
    ),
    scratch_types=[
        pltpu.VMEM((128,), jnp.int32),
        pltpu.VMEM((128, 128), jnp.float32),
        pltpu.VMEM_SHARED((ACC_ROWS, 128), jnp.float32),
    ],
    mesh=_mesh2,
)
def _deg_kernel(src1d, dst1d, ones_hbm, zeros_hbm, deg_out_hbm, deg_in_hbm,
                idxv, ones_v, acc):
    c = lax.axis_index("c")
    s = lax.axis_index("s")
    e0 = (c * NTILE + s) * DEG_ROWS_PER_TILE * 128
    o0 = s * ZROWS
    pltpu.sync_copy(ones_hbm, ones_v)
    for idx_hbm, out_hbm in ((src1d, deg_out_hbm), (dst1d, deg_in_hbm)):
        pltpu.sync_copy(zeros_hbm, acc.at[pl.ds(o0, ZROWS)])
        plsc.subcore_barrier()

        @pl.loop(0, DEG_ROWS_PER_TILE)
        def _(j, idx_hbm=idx_hbm):
            pltpu.sync_copy(idx_hbm.at[pl.ds(e0 + j * 128, 128)], idxv)
            pltpu.sync_copy(ones_v, acc.at[idxv], add=True)

        plsc.subcore_barrier()
        pltpu.sync_copy(acc.at[pl.ds(o0, ZROWS)],
                        out_hbm.at[c, pl.ds(o0, ZROWS)])
        plsc.subcore_barrier()


@functools.partial(
    pl.kernel,
    out_type=(
        jax.ShapeDtypeStruct((ACC_ROWS, 128), jnp.float32),
        jax.ShapeDtypeStruct((ACC_ROWS, 128), jnp.float32),
    ),
    scratch_types=[
        pltpu.VMEM((128,), jnp.int32),
        pltpu.VMEM((128,), jnp.int32),
        pltpu.VMEM((128, 128), jnp.float32),
        pltpu.VMEM((16, 128), jnp.float32),
        pltpu.VMEM((8, 128), jnp.float32),
        pltpu.VMEM((8, 128), jnp.float32),
        pltpu.VMEM((1, 128), jnp.float32),
        pltpu.VMEM_SHARED((ACC_ROWS, 128), jnp.float32),
        pltpu.SemaphoreType.DMA,
    ],
    mesh=_mesh1,
)
def _mega(src1d, dst1d, h1, ni_mat, no_mat, b1r, zeros_hbm,
          q_hbm, t2_hbm,
          sidx, didx, rows, ebuf, nibuf, nobuf, b1v, acc, sem):
    s = lax.axis_index("s")
    e0 = s * EROWS * 128
    base = s * ZROWS
    pltpu.sync_copy(zeros_hbm, acc.at[pl.ds(base, ZROWS)])
    pltpu.sync_copy(b1r, b1v)
    pltpu.sync_copy(ni_mat.at[pl.ds(s * 8, 8)], nibuf)
    pltpu.sync_copy(no_mat.at[pl.ds(s * 8, 8)], nobuf)

    plsc.subcore_barrier()

    def _scatter_pass(table):
        @pl.loop(0, EROWS)
        def _(i):
            pltpu.sync_copy(src1d.at[pl.ds(e0 + i * 128, 128)], sidx)
            pltpu.sync_copy(dst1d.at[pl.ds(e0 + i * 128, 128)], didx)
            pltpu.async_copy(table.at[sidx], rows, sem).wait()
            pltpu.sync_copy(rows, acc.at[didx], add=True)

    _scatter_pass(h1)
    plsc.subcore_barrier()

    b1g = [b1v[0, pl.ds(g * 16, 16)] for g in range(8)]
    for kb in range(5):
        @pl.loop(0, 8)
        def _(t, kb=kb):
            pltpu.sync_copy(acc.at[pl.ds(base + kb * 128 + t * 16, 16)], ebuf)
            niv = nibuf[kb, pl.ds(t * 16, 16)]
            nov = nobuf[kb, pl.ds(t * 16, 16)]
            for rr in range(16):
                for g in range(8):
                    v = ebuf[rr, pl.ds(g * 16, 16)]
                    v = jnp.maximum(v * niv[rr] + b1g[g], 0.0) * nov[rr]
                    ebuf[rr, pl.ds(g * 16, 16)] = v
            pltpu.sync_copy(ebuf,
                            t2_hbm.at[pl.ds(base + kb * 128 + t * 16, 16)])

    pltpu.sync_copy(zeros_hbm, acc.at[pl.ds(base, ZROWS)])
    plsc.subcore_barrier()

    _scatter_pass(t2_hbm)
    plsc.subcore_barrier()
    pltpu.sync_copy(acc.at[pl.ds(base, ZROWS)], q_hbm.at[pl.ds(base, ZROWS)])


BN1 = 512
BN2 = 400


def _mm1_body(x_ref, w_ref, dego_ref, o_ref):
    deg = dego_ref[0, :, 0] + dego_ref[1, :, 0]
    norm = lax.rsqrt(jnp.maximum(deg, 1.0))
    o_ref[...] = jnp.dot(x_ref[...] * norm[:, None], w_ref[...],
                         preferred_element_type=jnp.float32)


_mm1 = pl.pallas_call(
    _mm1_body,
    grid=(ACC_ROWS // BN1,),
    in_specs=[
        pl.BlockSpec((BN1, 128), lambda i: (i, 0)),
        pl.BlockSpec((128, 128), lambda i: (0, 0)),
        pl.BlockSpec((NCORE, BN1, 128), lambda i: (0, i, 0)),
    ],
    out_specs=pl.BlockSpec((BN1, 128), lambda i: (i, 0)),
    out_shape=jax.ShapeDtypeStruct((ACC_ROWS, 128), jnp.float32),
)


def _norm_body(degi_ref, dego_ref, ni_ref, no_ref):
    di = jnp.sum(degi_ref[0] + degi_ref[1], axis=-1) * (1.0 / 128.0)
    do = jnp.sum(dego_ref[0] + dego_ref[1], axis=-1) * (1.0 / 128.0)
    ni_ref[...] = lax.rsqrt(jnp.maximum(di, 1.0))
    no_ref[...] = lax.rsqrt(jnp.maximum(do, 1.0))


_norm = pl.pallas_call(
    _norm_body,
    out_shape=(
        jax.ShapeDtypeStruct((ACC_ROWS // 128, 128), jnp.float32),
        jax.ShapeDtypeStruct((ACC_ROWS // 128, 128), jnp.float32),
    ),
)


def _fin_body(q_ref, degi_ref, b2_ref, w2_ref, o_ref):
    ni = lax.rsqrt(jnp.maximum(degi_ref[0, :, 0] + degi_ref[1, :, 0], 1.0))
    o_ref[...] = jnp.dot(q_ref[...] * ni[:, None], w2_ref[...],
                         preferred_element_type=jnp.float32) + b2_ref[...]


_fin = pl.pallas_call(
    _fin_body,
    grid=(N // BN2,),
    in_specs=[
        pl.BlockSpec((BN2, 128), lambda i: (i, 0)),
        pl.BlockSpec((NCORE, BN2, 128), lambda i: (0, i, 0)),
        pl.BlockSpec((1, 64), lambda i: (0, 0)),
        pl.BlockSpec((128, 64), lambda i: (0, 0)),
    ],
    out_specs=pl.BlockSpec((BN2, 64), lambda i: (i, 0)),
    out_shape=jax.ShapeDtypeStruct((N, 64), jnp.float32),
)


def _tile_pad_fn(m):
    return jnp.pad(m.reshape(NTILE, 5, 128),
                   ((0, 0), (0, 3), (0, 0))).reshape(NTILE * 8, 128)


def kernel(x, edge_index, W1, b1, W2, b2):
    src = edge_index[0]
    dst = edge_index[1]
    padN = jnp.full((EPAD - E,), N, jnp.int32)
    src1d = jnp.concatenate([src, padN])
    dst1d = jnp.concatenate([dst, padN])
    x_pad = jnp.concatenate(
        [x, jnp.zeros((ACC_ROWS - N, 128), jnp.float32)])
    ones128 = jnp.ones((128, 128), jnp.float32)
    zeros128 = jnp.zeros((ZROWS, 128), jnp.float32)

    dego_p, degi_p = _deg_kernel(src1d, dst1d, ones128, zeros128)
    ni_mat, no_mat = _norm(degi_p.reshape(NCORE, ACC_ROWS // 128, 128, 128),
                           dego_p.reshape(NCORE, ACC_ROWS // 128, 128, 128))

    h1 = _mm1(x_pad, W1, dego_p)
    q2, _ = _mega(src1d, dst1d, h1, _tile_pad_fn(ni_mat), _tile_pad_fn(no_mat),
                  b1.reshape(1, -1), zeros128)
    return _fin(q2, degi_p, b2.reshape(1, -1), W2)


# speedup vs baseline: 1.9427x; 1.1180x over previous
import functools

import jax
import jax.numpy as jnp
from jax import lax
from jax.experimental import pallas as pl
from jax.experimental.pallas import tpu as pltpu
from jax.experimental.pallas import tpu_sc as plsc

N = 10000
E = 320000
NTILE = 16
NCORE = 2
EROWS = 160
EPAD = NTILE * EROWS * 128
ACC_ROWS = 10240
ZROWS = ACC_ROWS // NTILE
DEG_ROWS_PER_TILE = EPAD // (NCORE * NTILE * 128)
CH = 4

_mesh2 = plsc.VectorSubcoreMesh(core_axis_name="c", subcore_axis_name="s")
_mesh1 = plsc.VectorSubcoreMesh(core_axis_name="c", subcore_axis_name="s",
                                num_cores=1)


@functools.partial(
    pl.kernel,
    out_type=(
        jax.ShapeDtypeStruct((NCORE, ACC_ROWS, 128), jnp.float32),
        jax.ShapeDtypeStruct((NCORE, ACC_ROWS, 128), jnp.float32),
    ),
    scratch_types=[
        pltpu.VMEM((128,), jnp.int32),
        pltpu.VMEM((128, 128), jnp.float32),
        pltpu.VMEM_SHARED((ACC_ROWS, 128), jnp.float32),
    ],
    mesh=_mesh2,
)
def _deg_kernel(src1d, dst1d, ones_hbm, zeros_hbm, deg_out_hbm, deg_in_hbm,
                idxv, ones_v, acc):
    c = lax.axis_index("c")
    s = lax.axis_index("s")
    e0 = (c * NTILE + s) * DEG_ROWS_PER_TILE * 128
    o0 = s * ZROWS
    pltpu.sync_copy(ones_hbm, ones_v)
    for idx_hbm, out_hbm in ((src1d, deg_out_hbm), (dst1d, deg_in_hbm)):
        pltpu.sync_copy(zeros_hbm, acc.at[pl.ds(o0, ZROWS)])
        plsc.subcore_barrier()

        @pl.loop(0, DEG_ROWS_PER_TILE)
        def _(j, idx_hbm=idx_hbm):
            pltpu.sync_copy(idx_hbm.at[pl.ds(e0 + j * 128, 128)], idxv)
            pltpu.sync_copy(ones_v, acc.at[idxv], add=True)

        plsc.subcore_barrier()
        pltpu.sync_copy(acc.at[pl.ds(o0, ZROWS)],
                        out_hbm.at[c, pl.ds(o0, ZROWS)])
        plsc.subcore_barrier()


@functools.partial(
    pl.kernel,
    out_type=(
        jax.ShapeDtypeStruct((ACC_ROWS, 128), jnp.float32),
        jax.ShapeDtypeStruct((ACC_ROWS, 128), jnp.float32),
    ),
    scratch_types=[
        pltpu.VMEM((64,), jnp.int32),
        pltpu.VMEM((64,), jnp.int32),
        pltpu.VMEM((64,), jnp.int32),
        pltpu.VMEM((64,), jnp.int32),
        pltpu.VMEM((64, 128), jnp.float32),
        pltpu.VMEM((64, 128), jnp.float32),
        pltpu.VMEM((16, 128), jnp.float32),
        pltpu.VMEM((8, 128), jnp.float32),
        pltpu.VMEM((8, 128), jnp.float32),
        pltpu.VMEM((1, 128), jnp.float32),
        pltpu.VMEM_SHARED((ACC_ROWS, 128), jnp.float32),
        pltpu.SemaphoreType.DMA,
        pltpu.SemaphoreType.DMA,
    ],
    mesh=_mesh1,
)
def _mega(src1d, dst1d, h1, ni_mat, no_mat, b1r, zeros_hbm,
          q_hbm, t2_hbm,
          sidx0, didx0, sidx1, didx1, rows0, rows1,
          ebuf, nibuf, nobuf, b1v, acc, sem, sem2):
    s = lax.axis_index("s")
    e0 = s * EROWS * 128
    base = s * ZROWS
    pltpu.sync_copy(zeros_hbm, acc.at[pl.ds(base, ZROWS)])
    pltpu.sync_copy(b1r, b1v)
    pltpu.sync_copy(ni_mat.at[pl.ds(s * 8, 8)], nibuf)
    pltpu.sync_copy(no_mat.at[pl.ds(s * 8, 8)], nobuf)

    plsc.subcore_barrier()

    def _scatter_pass(table):
        @pl.loop(0, EROWS)
        def _(i):
            eo = e0 + i * 128
            pltpu.async_copy(src1d.at[pl.ds(eo, 64)], sidx0, sem2)
            pltpu.async_copy(dst1d.at[pl.ds(eo, 64)], didx0, sem2)
            pltpu.async_copy(src1d.at[pl.ds(eo + 64, 64)], sidx1, sem2)
            pltpu.async_copy(dst1d.at[pl.ds(eo + 64, 64)], didx1, sem2)
            pltpu.make_async_copy(src1d.at[pl.ds(eo, 64)], sidx0, sem2).wait()
            pltpu.make_async_copy(dst1d.at[pl.ds(eo, 64)], didx0, sem2).wait()
            pltpu.make_async_copy(src1d.at[pl.ds(eo + 64, 64)], sidx1,
                                  sem2).wait()
            pltpu.make_async_copy(dst1d.at[pl.ds(eo + 64, 64)], didx1,
                                  sem2).wait()
            cp0 = pltpu.async_copy(table.at[sidx0], rows0, sem)
            cp1 = pltpu.async_copy(table.at[sidx1], rows1, sem)
            cp0.wait()
            pltpu.sync_copy(rows0, acc.at[didx0], add=True)
            cp1.wait()
            pltpu.sync_copy(rows1, acc.at[didx1], add=True)

    _scatter_pass(h1)
    plsc.subcore_barrier()

    b1g = [b1v[0, pl.ds(g * 16, 16)] for g in range(8)]
    for kb in range(5):
        @pl.loop(0, 8)
        def _(t, kb=kb):
            pltpu.sync_copy(acc.at[pl.ds(base + kb * 128 + t * 16, 16)], ebuf)
            niv = nibuf[kb, pl.ds(t * 16, 16)]
            nov = nobuf[kb, pl.ds(t * 16, 16)]
            for rr in range(16):
                for g in range(8):
                    v = ebuf[rr, pl.ds(g * 16, 16)]
                    v = jnp.maximum(v * niv[rr] + b1g[g], 0.0) * nov[rr]
                    ebuf[rr, pl.ds(g * 16, 16)] = v
            pltpu.sync_copy(ebuf,
                            t2_hbm.at[pl.ds(base + kb * 128 + t * 16, 16)])

    pltpu.sync_copy(zeros_hbm, acc.at[pl.ds(base, ZROWS)])
    plsc.subcore_barrier()

    _scatter_pass(t2_hbm)
    plsc.subcore_barrier()
    pltpu.sync_copy(acc.at[pl.ds(base, ZROWS)], q_hbm.at[pl.ds(base, ZROWS)])


BN1 = 512
BN2 = 400


def _mm1_body(x_ref, w_ref, dego_ref, o_ref):
    deg = dego_ref[0, :, 0] + dego_ref[1, :, 0]
    norm = lax.rsqrt(jnp.maximum(deg, 1.0))
    o_ref[...] = jnp.dot(x_ref[...] * norm[:, None], w_ref[...],
                         preferred_element_type=jnp.float32)


_mm1 = pl.pallas_call(
    _mm1_body,
    grid=(ACC_ROWS // BN1,),
    in_specs=[
        pl.BlockSpec((BN1, 128), lambda i: (i, 0)),
        pl.BlockSpec((128, 128), lambda i: (0, 0)),
        pl.BlockSpec((NCORE, BN1, 128), lambda i: (0, i, 0)),
    ],
    out_specs=pl.BlockSpec((BN1, 128), lambda i: (i, 0)),
    out_shape=jax.ShapeDtypeStruct((ACC_ROWS, 128), jnp.float32),
)


def _norm_body(degi_ref, dego_ref, ni_ref, no_ref):
    di = jnp.sum(degi_ref[0] + degi_ref[1], axis=-1) * (1.0 / 128.0)
    do = jnp.sum(dego_ref[0] + dego_ref[1], axis=-1) * (1.0 / 128.0)
    ni_ref[...] = lax.rsqrt(jnp.maximum(di, 1.0))
    no_ref[...] = lax.rsqrt(jnp.maximum(do, 1.0))


_norm = pl.pallas_call(
    _norm_body,
    out_shape=(
        jax.ShapeDtypeStruct((ACC_ROWS // 128, 128), jnp.float32),
        jax.ShapeDtypeStruct((ACC_ROWS // 128, 128), jnp.float32),
    ),
)


def _fin_body(q_ref, degi_ref, b2_ref, w2_ref, o_ref):
    ni = lax.rsqrt(jnp.maximum(degi_ref[0, :, 0] + degi_ref[1, :, 0], 1.0))
    o_ref[...] = jnp.dot(q_ref[...] * ni[:, None], w2_ref[...],
                         preferred_element_type=jnp.float32) + b2_ref[...]


_fin = pl.pallas_call(
    _fin_body,
    grid=(N // BN2,),
    in_specs=[
        pl.BlockSpec((BN2, 128), lambda i: (i, 0)),
        pl.BlockSpec((NCORE, BN2, 128), lambda i: (0, i, 0)),
        pl.BlockSpec((1, 64), lambda i: (0, 0)),
        pl.BlockSpec((128, 64), lambda i: (0, 0)),
    ],
    out_specs=pl.BlockSpec((BN2, 64), lambda i: (i, 0)),
    out_shape=jax.ShapeDtypeStruct((N, 64), jnp.float32),
)


def _tile_pad_fn(m):
    return jnp.pad(m.reshape(NTILE, 5, 128),
                   ((0, 0), (0, 3), (0, 0))).reshape(NTILE * 8, 128)


def kernel(x, edge_index, W1, b1, W2, b2):
    src = edge_index[0]
    dst = edge_index[1]
    padN = jnp.full((EPAD - E,), N, jnp.int32)
    src1d = jnp.concatenate([src, padN])
    dst1d = jnp.concatenate([dst, padN])
    x_pad = jnp.concatenate(
        [x, jnp.zeros((ACC_ROWS - N, 128), jnp.float32)])
    ones128 = jnp.ones((128, 128), jnp.float32)
    zeros128 = jnp.zeros((ZROWS, 128), jnp.float32)

    dego_p, degi_p = _deg_kernel(src1d, dst1d, ones128, zeros128)
    ni_mat, no_mat = _norm(degi_p.reshape(NCORE, ACC_ROWS // 128, 128, 128),
                           dego_p.reshape(NCORE, ACC_ROWS // 128, 128, 128))

    h1 = _mm1(x_pad, W1, dego_p)
    q2, _ = _mega(src1d, dst1d, h1, _tile_pad_fn(ni_mat), _tile_pad_fn(no_mat),
                  b1.reshape(1, -1), zeros128)
    return _fin(q2, degi_p, b2.reshape(1, -1), W2)


# speedup vs baseline: 1.9439x; 1.0006x over previous
import functools

import jax
import jax.numpy as jnp
from jax import lax
from jax.experimental import pallas as pl
from jax.experimental.pallas import tpu as pltpu
from jax.experimental.pallas import tpu_sc as plsc

N = 10000
E = 320000
NTILE = 16
NCORE = 2
EROWS = 160
EPAD = NTILE * EROWS * 128
ACC_ROWS = 10240
ZROWS = ACC_ROWS // NTILE
DEG_ROWS_PER_TILE = EPAD // (NCORE * NTILE * 128)
CH = 4

_mesh2 = plsc.VectorSubcoreMesh(core_axis_name="c", subcore_axis_name="s")
_mesh1 = plsc.VectorSubcoreMesh(core_axis_name="c", subcore_axis_name="s",
                                num_cores=1)


@functools.partial(
    pl.kernel,
    out_type=(
        jax.ShapeDtypeStruct((NCORE, ACC_ROWS, 128), jnp.float32),
        jax.ShapeDtypeStruct((NCORE, ACC_ROWS, 128), jnp.float32),
    ),
    scratch_types=[
        pltpu.VMEM((128,), jnp.int32),
        pltpu.VMEM((128, 128), jnp.float32),
        pltpu.VMEM_SHARED((ACC_ROWS, 128), jnp.float32),
    ],
    mesh=_mesh2,
)
def _deg_kernel(src1d, dst1d, ones_hbm, zeros_hbm, deg_out_hbm, deg_in_hbm,
                idxv, ones_v, acc):
    c = lax.axis_index("c")
    s = lax.axis_index("s")
    e0 = (c * NTILE + s) * DEG_ROWS_PER_TILE * 128
    o0 = s * ZROWS
    pltpu.sync_copy(ones_hbm, ones_v)
    for idx_hbm, out_hbm in ((src1d, deg_out_hbm), (dst1d, deg_in_hbm)):
        pltpu.sync_copy(zeros_hbm, acc.at[pl.ds(o0, ZROWS)])
        plsc.subcore_barrier()

        @pl.loop(0, DEG_ROWS_PER_TILE)
        def _(j, idx_hbm=idx_hbm):
            pltpu.sync_copy(idx_hbm.at[pl.ds(e0 + j * 128, 128)], idxv)
            pltpu.sync_copy(ones_v, acc.at[idxv], add=True)

        plsc.subcore_barrier()
        pltpu.sync_copy(acc.at[pl.ds(o0, ZROWS)],
                        out_hbm.at[c, pl.ds(o0, ZROWS)])
        plsc.subcore_barrier()


@functools.partial(
    pl.kernel,
    out_type=(
        jax.ShapeDtypeStruct((ACC_ROWS, 128), jnp.float32),
        jax.ShapeDtypeStruct((ACC_ROWS, 128), jnp.float32),
    ),
    scratch_types=[
        pltpu.VMEM((64,), jnp.int32),
        pltpu.VMEM((64,), jnp.int32),
        pltpu.VMEM((64,), jnp.int32),
        pltpu.VMEM((64,), jnp.int32),
        pltpu.VMEM((64, 128), jnp.float32),
        pltpu.VMEM((64, 128), jnp.float32),
        pltpu.VMEM((32, 128), jnp.float32),
        pltpu.VMEM((8, 128), jnp.float32),
        pltpu.VMEM((8, 128), jnp.float32),
        pltpu.VMEM((1, 128), jnp.float32),
        pltpu.VMEM_SHARED((ACC_ROWS, 128), jnp.float32),
        pltpu.SemaphoreType.DMA,
        pltpu.SemaphoreType.DMA,
    ],
    mesh=_mesh1,
)
def _mega(src1d, dst1d, h1, ni_mat, no_mat, b1r, zeros_hbm,
          q_hbm, t2_hbm,
          sidx0, didx0, sidx1, didx1, rows0, rows1,
          ebuf, nibuf, nobuf, b1v, acc, sem, sem2):
    s = lax.axis_index("s")
    e0 = s * EROWS * 128
    base = s * ZROWS
    pltpu.sync_copy(zeros_hbm, acc.at[pl.ds(base, ZROWS)])
    pltpu.sync_copy(b1r, b1v)
    pltpu.sync_copy(ni_mat.at[pl.ds(s * 8, 8)], nibuf)
    pltpu.sync_copy(no_mat.at[pl.ds(s * 8, 8)], nobuf)

    plsc.subcore_barrier()

    def _scatter_pass(table):
        @pl.loop(0, EROWS)
        def _(i):
            eo = e0 + i * 128
            pltpu.async_copy(src1d.at[pl.ds(eo, 64)], sidx0, sem2)
            pltpu.async_copy(dst1d.at[pl.ds(eo, 64)], didx0, sem2)
            pltpu.async_copy(src1d.at[pl.ds(eo + 64, 64)], sidx1, sem2)
            pltpu.async_copy(dst1d.at[pl.ds(eo + 64, 64)], didx1, sem2)
            pltpu.make_async_copy(src1d.at[pl.ds(eo, 64)], sidx0, sem2).wait()
            pltpu.make_async_copy(dst1d.at[pl.ds(eo, 64)], didx0, sem2).wait()
            pltpu.make_async_copy(src1d.at[pl.ds(eo + 64, 64)], sidx1,
                                  sem2).wait()
            pltpu.make_async_copy(dst1d.at[pl.ds(eo + 64, 64)], didx1,
                                  sem2).wait()
            cp0 = pltpu.async_copy(table.at[sidx0], rows0, sem)
            cp1 = pltpu.async_copy(table.at[sidx1], rows1, sem)
            cp0.wait()
            pltpu.sync_copy(rows0, acc.at[didx0], add=True)
            cp1.wait()
            pltpu.sync_copy(rows1, acc.at[didx1], add=True)

    _scatter_pass(h1)
    plsc.subcore_barrier()

    b1g = [b1v[0, pl.ds(g * 16, 16)] for g in range(8)]
    for kb in range(5):
        @pl.loop(0, 4)
        def _(t, kb=kb):
            pltpu.sync_copy(acc.at[pl.ds(base + kb * 128 + t * 32, 32)], ebuf)
            niv0 = nibuf[kb, pl.ds(t * 32, 16)]
            nov0 = nobuf[kb, pl.ds(t * 32, 16)]
            niv1 = nibuf[kb, pl.ds(t * 32 + 16, 16)]
            nov1 = nobuf[kb, pl.ds(t * 32 + 16, 16)]
            for half, niv, nov in ((0, niv0, nov0), (1, niv1, nov1)):
                for rr in range(16):
                    for g in range(8):
                        r = half * 16 + rr
                        v = ebuf[r, pl.ds(g * 16, 16)]
                        v = jnp.maximum(v * niv[rr] + b1g[g], 0.0) * nov[rr]
                        ebuf[r, pl.ds(g * 16, 16)] = v
            pltpu.sync_copy(ebuf,
                            t2_hbm.at[pl.ds(base + kb * 128 + t * 32, 32)])

    pltpu.sync_copy(zeros_hbm, acc.at[pl.ds(base, ZROWS)])
    plsc.subcore_barrier()

    _scatter_pass(t2_hbm)
    plsc.subcore_barrier()
    pltpu.sync_copy(acc.at[pl.ds(base, ZROWS)], q_hbm.at[pl.ds(base, ZROWS)])


BN1 = 512
BN2 = 400


def _mm1_body(x_ref, w_ref, dego_ref, o_ref):
    deg = dego_ref[0, :, 0] + dego_ref[1, :, 0]
    norm = lax.rsqrt(jnp.maximum(deg, 1.0))
    o_ref[...] = jnp.dot(x_ref[...] * norm[:, None], w_ref[...],
                         preferred_element_type=jnp.float32)


_mm1 = pl.pallas_call(
    _mm1_body,
    grid=(ACC_ROWS // BN1,),
    in_specs=[
        pl.BlockSpec((BN1, 128), lambda i: (i, 0)),
        pl.BlockSpec((128, 128), lambda i: (0, 0)),
        pl.BlockSpec((NCORE, BN1, 128), lambda i: (0, i, 0)),
    ],
    out_specs=pl.BlockSpec((BN1, 128), lambda i: (i, 0)),
    out_shape=jax.ShapeDtypeStruct((ACC_ROWS, 128), jnp.float32),
)


def _norm_body(degi_ref, dego_ref, ni_ref, no_ref):
    di = jnp.sum(degi_ref[0] + degi_ref[1], axis=-1) * (1.0 / 128.0)
    do = jnp.sum(dego_ref[0] + dego_ref[1], axis=-1) * (1.0 / 128.0)
    ni_ref[...] = lax.rsqrt(jnp.maximum(di, 1.0))
    no_ref[...] = lax.rsqrt(jnp.maximum(do, 1.0))


_norm = pl.pallas_call(
    _norm_body,
    out_shape=(
        jax.ShapeDtypeStruct((ACC_ROWS // 128, 128), jnp.float32),
        jax.ShapeDtypeStruct((ACC_ROWS // 128, 128), jnp.float32),
    ),
)


def _fin_body(q_ref, degi_ref, b2_ref, w2_ref, o_ref):
    ni = lax.rsqrt(jnp.maximum(degi_ref[0, :, 0] + degi_ref[1, :, 0], 1.0))
    o_ref[...] = jnp.dot(q_ref[...] * ni[:, None], w2_ref[...],
                         preferred_element_type=jnp.float32) + b2_ref[...]


_fin = pl.pallas_call(
    _fin_body,
    grid=(N // BN2,),
    in_specs=[
        pl.BlockSpec((BN2, 128), lambda i: (i, 0)),
        pl.BlockSpec((NCORE, BN2, 128), lambda i: (0, i, 0)),
        pl.BlockSpec((1, 64), lambda i: (0, 0)),
        pl.BlockSpec((128, 64), lambda i: (0, 0)),
    ],
    out_specs=pl.BlockSpec((BN2, 64), lambda i: (i, 0)),
    out_shape=jax.ShapeDtypeStruct((N, 64), jnp.float32),
)


def _tile_pad_fn(m):
    return jnp.pad(m.reshape(NTILE, 5, 128),
                   ((0, 0), (0, 3), (0, 0))).reshape(NTILE * 8, 128)


def kernel(x, edge_index, W1, b1, W2, b2):
    src = edge_index[0]
    dst = edge_index[1]
    padN = jnp.full((EPAD - E,), N, jnp.int32)
    src1d = jnp.concatenate([src, padN])
    dst1d = jnp.concatenate([dst, padN])
    x_pad = jnp.concatenate(
        [x, jnp.zeros((ACC_ROWS - N, 128), jnp.float32)])
    ones128 = jnp.ones((128, 128), jnp.float32)
    zeros128 = jnp.zeros((ZROWS, 128), jnp.float32)

    dego_p, degi_p = _deg_kernel(src1d, dst1d, ones128, zeros128)
    ni_mat, no_mat = _norm(degi_p.reshape(NCORE, ACC_ROWS // 128, 128, 128),
                           dego_p.reshape(NCORE, ACC_ROWS // 128, 128, 128))

    h1 = _mm1(x_pad, W1, dego_p)
    q2, _ = _mega(src1d, dst1d, h1, _tile_pad_fn(ni_mat), _tile_pad_fn(no_mat),
                  b1.reshape(1, -1), zeros128)
    return _fin(q2, degi_p, b2.reshape(1, -1), W2)
